# baseline (speedup 1.0000x reference)
import functools

import jax
import jax.numpy as jnp
from jax import lax
from jax.experimental import pallas as pl
from jax.experimental.pallas import tpu as pltpu
from jax.experimental.pallas import tpu_sc as plsc

TEMP = 0.07
BASE_TEMP = 0.07
N_MEM = 50020
BSZ = 4096
DFEAT = 64
NCON = 2 * BSZ

BLK = 128
NBLK = BSZ // BLK
DUMMY = 1 << 20

NC, NS = 2, 16
NW = NC * NS
G_CHUNK = BSZ // NW
LANES = 16
SHARD = 3128
LAST_START = (NS - 1) * SHARD
LAST_N = N_MEM - LAST_START


def _dense_body(a_ref, ct_ref, b_ref, idxc_ref, ug_ref, f1_ref, vg_ref,
                f2_ref, ugall_ref, unew_ref, vnew_ref, rsum_ref, mpp_ref,
                idxs_ref, loss_ref, acc_ref):
    b = pl.program_id(0)
    a = a_ref[...] * (1.0 / TEMP)
    logits = jnp.dot(a, ct_ref[...], preferred_element_type=jnp.float32)
    m = jnp.max(logits, axis=1, keepdims=True)
    e = jnp.exp(logits - m)

    col = lax.broadcasted_iota(jnp.int32, (BLK, NCON), 1)
    rowg = b * BLK + lax.broadcasted_iota(jnp.int32, (BLK, NCON), 0)
    row_sum = jnp.sum(jnp.where(col == 2 * rowg, 0.0, e), axis=1,
                      keepdims=True)
    pos_l = jnp.sum(a * b_ref[...], axis=1, keepdims=True)
    pos_e = jnp.exp(pos_l - m)

    gs = jnp.sum(ugall_ref[...])
    gamma = jnp.where(gs == 0.0, 1.0, 0.9)
    omg = 1.0 - gamma
    u_new = omg * (ug_ref[...] - f1_ref[...]) + row_sum
    mpp = pos_e / u_new
    v_new = omg * (vg_ref[...] - f2_ref[...]) + mpp

    idxb = idxc_ref[...]
    lane = lax.broadcasted_iota(jnp.int32, (BLK, 1), 0) % LANES
    loser = jnp.zeros((BLK, 1), jnp.bool_)
    for t in range(1, LANES):
        nb = jnp.concatenate(
            [idxb[t:], jnp.full((t, 1), -1, jnp.int32)], axis=0)
        loser = loser | ((idxb == nb) & (lane < LANES - t))
    idxs_ref[...] = jnp.where(loser, DUMMY, idxb)

    unew_ref[...] = u_new
    vnew_ref[...] = v_new
    rsum_ref[...] = row_sum
    mpp_ref[...] = mpp

    part = jnp.sum(jnp.log(v_new))
    prev = jnp.where(b == 0, 0.0, acc_ref[0])
    acc_ref[0] = prev + part

    @pl.when(b == NBLK - 1)
    def _():
        val = -(TEMP / BASE_TEMP) * acc_ref[0] / BSZ
        loss_ref[...] = jnp.reshape(val, (1, 1))


@functools.cache
def _make_dense(interpret=False):
    col = lambda b: (b, 0)
    full = lambda b: (0, 0)
    return pl.pallas_call(
        _dense_body,
        grid=(NBLK,),
        in_specs=[
            pl.BlockSpec((BLK, DFEAT), col),
            pl.BlockSpec((DFEAT, NCON), full),
            pl.BlockSpec((BLK, DFEAT), col),
            pl.BlockSpec((BLK, 1), col),
            pl.BlockSpec((BLK, 1), col),
            pl.BlockSpec((BLK, 1), col),
            pl.BlockSpec((BLK, 1), col),
            pl.BlockSpec((BLK, 1), col),
            pl.BlockSpec((1, BSZ), full),
        ],
        out_specs=[
            pl.BlockSpec((BLK, 1), col),
            pl.BlockSpec((BLK, 1), col),
            pl.BlockSpec((BLK, 1), col),
            pl.BlockSpec((BLK, 1), col),
            pl.BlockSpec((BLK, 1), col),
            pl.BlockSpec((1, 1), full),
        ],
        out_shape=[
            jax.ShapeDtypeStruct((BSZ, 1), jnp.float32),
            jax.ShapeDtypeStruct((BSZ, 1), jnp.float32),
            jax.ShapeDtypeStruct((BSZ, 1), jnp.float32),
            jax.ShapeDtypeStruct((BSZ, 1), jnp.float32),
            jax.ShapeDtypeStruct((BSZ, 1), jnp.int32),
            jax.ShapeDtypeStruct((1, 1), jnp.float32),
        ],
        scratch_shapes=[pltpu.SMEM((1,), jnp.float32)],
        interpret=interpret,
    )


@functools.cache
def _make_sc_gather():
    mesh = plsc.VectorSubcoreMesh(core_axis_name="c", subcore_axis_name="s",
                                  num_cores=NC, num_subcores=NS)

    @functools.partial(
        pl.kernel, mesh=mesh,
        out_type=[jax.ShapeDtypeStruct((BSZ,), jnp.float32)] * 4,
        scratch_types=[pltpu.VMEM((G_CHUNK,), jnp.int32)]
        + [pltpu.VMEM((G_CHUNK,), jnp.float32)] * 4
        + [pltpu.SemaphoreType.DMA],
    )
    def sc_gather(idx_hbm, u_hbm, f1_hbm, v_hbm, f2_hbm,
                  ug_out, f1g_out, vg_out, f2g_out,
                  idx_v, b0, b1, b2, b3, sem):
        wid = lax.axis_index("s") * NC + lax.axis_index("c")
        base = wid * G_CHUNK
        pltpu.sync_copy(idx_hbm.at[pl.ds(base, G_CHUNK)], idx_v)
        c0 = pltpu.async_copy(u_hbm.at[idx_v], b0, sem)
        c1 = pltpu.async_copy(f1_hbm.at[idx_v], b1, sem)
        c2 = pltpu.async_copy(v_hbm.at[idx_v], b2, sem)
        c3 = pltpu.async_copy(f2_hbm.at[idx_v], b3, sem)
        c0.wait(); c1.wait(); c2.wait(); c3.wait()
        pltpu.sync_copy(b0, ug_out.at[pl.ds(base, G_CHUNK)])
        pltpu.sync_copy(b1, f1g_out.at[pl.ds(base, G_CHUNK)])
        pltpu.sync_copy(b2, vg_out.at[pl.ds(base, G_CHUNK)])
        pltpu.sync_copy(b3, f2g_out.at[pl.ds(base, G_CHUNK)])

    return sc_gather


@functools.cache
def _make_sc_scatter():
    mesh = plsc.VectorSubcoreMesh(core_axis_name="c", subcore_axis_name="s",
                                  num_cores=NC, num_subcores=NS)

    @functools.partial(
        pl.kernel, mesh=mesh,
        out_type=[jax.ShapeDtypeStruct((N_MEM,), jnp.float32)] * 4,
        compiler_params=pltpu.CompilerParams(needs_layout_passes=False),
        scratch_types=[
            pltpu.VMEM((SHARD,), jnp.float32),
            pltpu.VMEM((SHARD,), jnp.float32),
            pltpu.VMEM((BSZ,), jnp.int32),
            pltpu.VMEM((BSZ,), jnp.float32),
            pltpu.VMEM((BSZ,), jnp.float32),
            pltpu.SemaphoreType.DMA,
        ],
    )
    def sc_scatter(u_hbm, f1_hbm, v_hbm, f2_hbm, idx_hbm,
                   unew_hbm, rsum_hbm, vnew_hbm, mpp_hbm,
                   u_out, f1_out, v_out, f2_out,
                   t0_v, t1_v, idx_v, val0_v, val1_v, sem):
        cid = lax.axis_index("c")
        sid = lax.axis_index("s")
        lo = sid * SHARD

        def stage(src0, src1):
            cs = [pltpu.async_copy(idx_hbm, idx_v, sem),
                  pltpu.async_copy(src0, val0_v, sem),
                  pltpu.async_copy(src1, val1_v, sem)]
            return cs

        def shards_in(b0, b1):
            @pl.when(sid < NS - 1)
            def _():
                pltpu.sync_copy(b0.at[pl.ds(sid * SHARD, SHARD)], t0_v)
                pltpu.sync_copy(b1.at[pl.ds(sid * SHARD, SHARD)], t1_v)

            @pl.when(sid == NS - 1)
            def _():
                pltpu.sync_copy(b0.at[pl.ds(LAST_START, LAST_N)],
                                t0_v.at[pl.ds(0, LAST_N)])
                pltpu.sync_copy(b1.at[pl.ds(LAST_START, LAST_N)],
                                t1_v.at[pl.ds(0, LAST_N)])

        def shards_out(o0, o1):
            @pl.when(sid < NS - 1)
            def _():
                pltpu.sync_copy(t0_v, o0.at[pl.ds(sid * SHARD, SHARD)])
                pltpu.sync_copy(t1_v, o1.at[pl.ds(sid * SHARD, SHARD)])

            @pl.when(sid == NS - 1)
            def _():
                pltpu.sync_copy(t0_v.at[pl.ds(0, LAST_N)],
                                o0.at[pl.ds(LAST_START, LAST_N)])
                pltpu.sync_copy(t1_v.at[pl.ds(0, LAST_N)],
                                o1.at[pl.ds(LAST_START, LAST_N)])

        def apply_updates():
            def body(k, _):
                iv = idx_v[pl.ds(k * LANES, LANES)]
                msk = (iv >= lo) & (iv < lo + SHARD)
                li = jnp.where(msk, iv - lo, 0)
                plsc.store_scatter(t0_v, [li],
                                   val0_v[pl.ds(k * LANES, LANES)], mask=msk)
                plsc.store_scatter(t1_v, [li],
                                   val1_v[pl.ds(k * LANES, LANES)], mask=msk)
                return 0
            lax.fori_loop(0, BSZ // LANES, body, 0)

        @pl.when(cid == 0)
        def _():
            cs = stage(unew_hbm, rsum_hbm)
            shards_in(u_hbm, f1_hbm)
            for c in cs:
                c.wait()
            apply_updates()
            shards_out(u_out, f1_out)

        @pl.when(cid == 1)
        def _():
            cs = stage(vnew_hbm, mpp_hbm)
            shards_in(v_hbm, f2_hbm)
            for c in cs:
                c.wait()
            apply_updates()
            shards_out(v_out, f2_out)

    return sc_scatter


def kernel(features, index, u, v, f1_w, f2_w):
    anchor = features[:, 0, :]
    ct = features.reshape(NCON, DFEAT).T
    idx = index.astype(jnp.int32)
    u_flat = u.reshape(N_MEM)
    f1_flat = f1_w.reshape(N_MEM)

    ug, f1g, vg, f2g = _make_sc_gather()(idx, u_flat, f1_flat, v, f2_w)

    unew, vnew, rsum, mpp, idxs, loss = _make_dense()(
        anchor, ct, features[:, 1, :],
        idx.reshape(BSZ, 1),
        ug.reshape(BSZ, 1), f1g.reshape(BSZ, 1),
        vg.reshape(BSZ, 1), f2g.reshape(BSZ, 1),
        ug.reshape(1, BSZ),
    )

    u_out, f1_out, v_out, f2_out = _make_sc_scatter()(
        u_flat, f1_flat, v, f2_w, idxs.reshape(BSZ),
        unew.reshape(BSZ), rsum.reshape(BSZ),
        vnew.reshape(BSZ), mpp.reshape(BSZ),
    )

    return (loss[0, 0],
            u_out.reshape(N_MEM, 1),
            v_out,
            f1_out.reshape(N_MEM, 1),
            f2_out)


# speedup vs baseline: 1.1400x; 1.1400x over previous
import functools

import jax
import jax.numpy as jnp
from jax import lax
from jax.experimental import pallas as pl
from jax.experimental.pallas import tpu as pltpu
from jax.experimental.pallas import tpu_sc as plsc

TEMP = 0.07
BASE_TEMP = 0.07
N_MEM = 50020
BSZ = 4096
DFEAT = 64
NCON = 2 * BSZ

BLK = 256
NBLK = BSZ // BLK
DUMMY = 1 << 20

NC, NS = 2, 16
NW = NC * NS
G_CHUNK = BSZ // NW
LANES = 16
SHARD = 3128
LAST_START = (NS - 1) * SHARD
LAST_N = N_MEM - LAST_START


def _dense_body(a_ref, ct_ref, b_ref, idxc_ref, ug_ref, f1_ref, vg_ref,
                f2_ref, ugall_ref, unew_ref, vnew_ref, rsum_ref, mpp_ref,
                idxs_ref, loss_ref, acc_ref):
    b = pl.program_id(0)
    a = a_ref[...] * (1.0 / TEMP)
    logits = jnp.dot(a, ct_ref[...], preferred_element_type=jnp.float32)
    m = jnp.max(logits, axis=1, keepdims=True)
    e = jnp.exp(logits - m)

    col = lax.broadcasted_iota(jnp.int32, (BLK, NCON), 1)
    rowg = b * BLK + lax.broadcasted_iota(jnp.int32, (BLK, NCON), 0)
    row_sum = jnp.sum(jnp.where(col == rowg, 0.0, e), axis=1, keepdims=True)
    pos_l = jnp.sum(a * b_ref[...], axis=1, keepdims=True)
    pos_e = jnp.exp(pos_l - m)

    gs = jnp.sum(ugall_ref[...])
    gamma = jnp.where(gs == 0.0, 1.0, 0.9)
    omg = 1.0 - gamma
    u_new = omg * (ug_ref[...] - f1_ref[...]) + row_sum
    mpp = pos_e / u_new
    v_new = omg * (vg_ref[...] - f2_ref[...]) + mpp

    idxb = idxc_ref[...]
    lane = lax.broadcasted_iota(jnp.int32, (BLK, 1), 0) % LANES
    loser = jnp.zeros((BLK, 1), jnp.bool_)
    for t in range(1, LANES):
        nb = jnp.concatenate(
            [idxb[t:], jnp.full((t, 1), -1, jnp.int32)], axis=0)
        loser = loser | ((idxb == nb) & (lane < LANES - t))
    idxs_ref[...] = jnp.where(loser, DUMMY, idxb)

    unew_ref[...] = u_new
    vnew_ref[...] = v_new
    rsum_ref[...] = row_sum
    mpp_ref[...] = mpp

    part = jnp.sum(jnp.log(v_new))
    prev = jnp.where(b == 0, 0.0, acc_ref[0])
    acc_ref[0] = prev + part

    @pl.when(b == NBLK - 1)
    def _():
        val = -(TEMP / BASE_TEMP) * acc_ref[0] / BSZ
        loss_ref[...] = jnp.reshape(val, (1, 1))


@functools.cache
def _make_dense(interpret=False):
    col = lambda b: (b, 0)
    full = lambda b: (0, 0)
    return pl.pallas_call(
        _dense_body,
        grid=(NBLK,),
        in_specs=[
            pl.BlockSpec((BLK, DFEAT), col),
            pl.BlockSpec((DFEAT, NCON), full),
            pl.BlockSpec((BLK, DFEAT), col),
            pl.BlockSpec((BLK, 1), col),
            pl.BlockSpec((BLK, 1), col),
            pl.BlockSpec((BLK, 1), col),
            pl.BlockSpec((BLK, 1), col),
            pl.BlockSpec((BLK, 1), col),
            pl.BlockSpec((1, BSZ), full),
        ],
        out_specs=[
            pl.BlockSpec((BLK, 1), col),
            pl.BlockSpec((BLK, 1), col),
            pl.BlockSpec((BLK, 1), col),
            pl.BlockSpec((BLK, 1), col),
            pl.BlockSpec((BLK, 1), col),
            pl.BlockSpec((1, 1), full),
        ],
        out_shape=[
            jax.ShapeDtypeStruct((BSZ, 1), jnp.float32),
            jax.ShapeDtypeStruct((BSZ, 1), jnp.float32),
            jax.ShapeDtypeStruct((BSZ, 1), jnp.float32),
            jax.ShapeDtypeStruct((BSZ, 1), jnp.float32),
            jax.ShapeDtypeStruct((BSZ, 1), jnp.int32),
            jax.ShapeDtypeStruct((1, 1), jnp.float32),
        ],
        scratch_shapes=[pltpu.SMEM((1,), jnp.float32)],
        interpret=interpret,
    )


@functools.cache
def _make_sc_gather():
    mesh = plsc.VectorSubcoreMesh(core_axis_name="c", subcore_axis_name="s",
                                  num_cores=NC, num_subcores=NS)

    @functools.partial(
        pl.kernel, mesh=mesh,
        out_type=[jax.ShapeDtypeStruct((BSZ,), jnp.float32)] * 4,
        scratch_types=[pltpu.VMEM((G_CHUNK,), jnp.int32)]
        + [pltpu.VMEM((G_CHUNK,), jnp.float32)] * 4
        + [pltpu.SemaphoreType.DMA],
    )
    def sc_gather(idx_hbm, u_hbm, f1_hbm, v_hbm, f2_hbm,
                  ug_out, f1g_out, vg_out, f2g_out,
                  idx_v, b0, b1, b2, b3, sem):
        wid = lax.axis_index("s") * NC + lax.axis_index("c")
        base = wid * G_CHUNK
        pltpu.sync_copy(idx_hbm.at[pl.ds(base, G_CHUNK)], idx_v)
        c0 = pltpu.async_copy(u_hbm.at[idx_v], b0, sem)
        c1 = pltpu.async_copy(f1_hbm.at[idx_v], b1, sem)
        c2 = pltpu.async_copy(v_hbm.at[idx_v], b2, sem)
        c3 = pltpu.async_copy(f2_hbm.at[idx_v], b3, sem)
        c0.wait(); c1.wait(); c2.wait(); c3.wait()
        pltpu.sync_copy(b0, ug_out.at[pl.ds(base, G_CHUNK)])
        pltpu.sync_copy(b1, f1g_out.at[pl.ds(base, G_CHUNK)])
        pltpu.sync_copy(b2, vg_out.at[pl.ds(base, G_CHUNK)])
        pltpu.sync_copy(b3, f2g_out.at[pl.ds(base, G_CHUNK)])

    return sc_gather


@functools.cache
def _make_sc_scatter():
    mesh = plsc.VectorSubcoreMesh(core_axis_name="c", subcore_axis_name="s",
                                  num_cores=NC, num_subcores=NS)

    @functools.partial(
        pl.kernel, mesh=mesh,
        out_type=[jax.ShapeDtypeStruct((N_MEM,), jnp.float32)] * 4,
        compiler_params=pltpu.CompilerParams(needs_layout_passes=False),
        scratch_types=[
            pltpu.VMEM((SHARD,), jnp.float32),
            pltpu.VMEM((SHARD,), jnp.float32),
            pltpu.VMEM((BSZ,), jnp.int32),
            pltpu.VMEM((BSZ,), jnp.float32),
            pltpu.VMEM((BSZ,), jnp.float32),
            pltpu.SemaphoreType.DMA,
        ],
    )
    def sc_scatter(u_hbm, f1_hbm, v_hbm, f2_hbm, idx_hbm,
                   unew_hbm, rsum_hbm, vnew_hbm, mpp_hbm,
                   u_out, f1_out, v_out, f2_out,
                   t0_v, t1_v, idx_v, val0_v, val1_v, sem):
        cid = lax.axis_index("c")
        sid = lax.axis_index("s")
        lo = sid * SHARD

        def stage(src0, src1):
            cs = [pltpu.async_copy(idx_hbm, idx_v, sem),
                  pltpu.async_copy(src0, val0_v, sem),
                  pltpu.async_copy(src1, val1_v, sem)]
            return cs

        def shards_in(b0, b1):
            @pl.when(sid < NS - 1)
            def _():
                pltpu.sync_copy(b0.at[pl.ds(sid * SHARD, SHARD)], t0_v)
                pltpu.sync_copy(b1.at[pl.ds(sid * SHARD, SHARD)], t1_v)

            @pl.when(sid == NS - 1)
            def _():
                pltpu.sync_copy(b0.at[pl.ds(LAST_START, LAST_N)],
                                t0_v.at[pl.ds(0, LAST_N)])
                pltpu.sync_copy(b1.at[pl.ds(LAST_START, LAST_N)],
                                t1_v.at[pl.ds(0, LAST_N)])

        def shards_out(o0, o1):
            @pl.when(sid < NS - 1)
            def _():
                pltpu.sync_copy(t0_v, o0.at[pl.ds(sid * SHARD, SHARD)])
                pltpu.sync_copy(t1_v, o1.at[pl.ds(sid * SHARD, SHARD)])

            @pl.when(sid == NS - 1)
            def _():
                pltpu.sync_copy(t0_v.at[pl.ds(0, LAST_N)],
                                o0.at[pl.ds(LAST_START, LAST_N)])
                pltpu.sync_copy(t1_v.at[pl.ds(0, LAST_N)],
                                o1.at[pl.ds(LAST_START, LAST_N)])

        def apply_updates():
            def body(k, _):
                iv = idx_v[pl.ds(k * LANES, LANES)]
                msk = (iv >= lo) & (iv < lo + SHARD)
                li = jnp.where(msk, iv - lo, 0)
                plsc.store_scatter(t0_v, [li],
                                   val0_v[pl.ds(k * LANES, LANES)], mask=msk)
                plsc.store_scatter(t1_v, [li],
                                   val1_v[pl.ds(k * LANES, LANES)], mask=msk)
                return 0
            lax.fori_loop(0, BSZ // LANES, body, 0)

        @pl.when(cid == 0)
        def _():
            cs = stage(unew_hbm, rsum_hbm)
            shards_in(u_hbm, f1_hbm)
            for c in cs:
                c.wait()
            apply_updates()
            shards_out(u_out, f1_out)

        @pl.when(cid == 1)
        def _():
            cs = stage(vnew_hbm, mpp_hbm)
            shards_in(v_hbm, f2_hbm)
            for c in cs:
                c.wait()
            apply_updates()
            shards_out(v_out, f2_out)

    return sc_scatter


def kernel(features, index, u, v, f1_w, f2_w):
    anchor = features[:, 0, :]
    ct = jnp.concatenate([anchor, features[:, 1, :]], axis=0).T
    idx = index.astype(jnp.int32)
    u_flat = u.reshape(N_MEM)
    f1_flat = f1_w.reshape(N_MEM)

    ug, f1g, vg, f2g = _make_sc_gather()(idx, u_flat, f1_flat, v, f2_w)

    unew, vnew, rsum, mpp, idxs, loss = _make_dense()(
        anchor, ct, features[:, 1, :],
        idx.reshape(BSZ, 1),
        ug.reshape(BSZ, 1), f1g.reshape(BSZ, 1),
        vg.reshape(BSZ, 1), f2g.reshape(BSZ, 1),
        ug.reshape(1, BSZ),
    )

    u_out, f1_out, v_out, f2_out = _make_sc_scatter()(
        u_flat, f1_flat, v, f2_w, idxs.reshape(BSZ),
        unew.reshape(BSZ), rsum.reshape(BSZ),
        vnew.reshape(BSZ), mpp.reshape(BSZ),
    )

    return (loss[0, 0],
            u_out.reshape(N_MEM, 1),
            v_out,
            f1_out.reshape(N_MEM, 1),
            f2_out)


# speedup vs baseline: 1.1726x; 1.0286x over previous
import functools

import jax
import jax.numpy as jnp
from jax import lax
from jax.experimental import pallas as pl
from jax.experimental.pallas import tpu as pltpu
from jax.experimental.pallas import tpu_sc as plsc

TEMP = 0.07
BASE_TEMP = 0.07
N_MEM = 50020
BSZ = 4096
DFEAT = 64
NCON = 2 * BSZ

BLK = 512
NBLK = BSZ // BLK
DUMMY = 1 << 20

NC, NS = 2, 16
NW = NC * NS
G_CHUNK = BSZ // NW
LANES = 16
SHARD = 3128
LAST_START = (NS - 1) * SHARD
LAST_N = N_MEM - LAST_START


def _dense_body(a_ref, ct_ref, b_ref, idxc_ref, ug_ref, f1_ref, vg_ref,
                f2_ref, ugall_ref, unew_ref, vnew_ref, rsum_ref, mpp_ref,
                idxs_ref, loss_ref, acc_ref):
    b = pl.program_id(0)
    a = a_ref[...] * (1.0 / TEMP)
    logits = jnp.dot(a, ct_ref[...], preferred_element_type=jnp.float32)
    m = jnp.max(logits, axis=1, keepdims=True)
    e = jnp.exp(logits - m)

    col = lax.broadcasted_iota(jnp.int32, (BLK, NCON), 1)
    rowg = b * BLK + lax.broadcasted_iota(jnp.int32, (BLK, NCON), 0)
    row_sum = jnp.sum(jnp.where(col == rowg, 0.0, e), axis=1, keepdims=True)
    pos_l = jnp.sum(a * b_ref[...], axis=1, keepdims=True)
    pos_e = jnp.exp(pos_l - m)

    gs = jnp.sum(ugall_ref[...])
    gamma = jnp.where(gs == 0.0, 1.0, 0.9)
    omg = 1.0 - gamma
    u_new = omg * (ug_ref[...] - f1_ref[...]) + row_sum
    mpp = pos_e / u_new
    v_new = omg * (vg_ref[...] - f2_ref[...]) + mpp

    idxb = idxc_ref[...]
    lane = lax.broadcasted_iota(jnp.int32, (BLK, 1), 0) % LANES
    loser = jnp.zeros((BLK, 1), jnp.bool_)
    for t in range(1, LANES):
        nb = jnp.concatenate(
            [idxb[t:], jnp.full((t, 1), -1, jnp.int32)], axis=0)
        loser = loser | ((idxb == nb) & (lane < LANES - t))
    idxs_ref[...] = jnp.where(loser, DUMMY, idxb)

    unew_ref[...] = u_new
    vnew_ref[...] = v_new
    rsum_ref[...] = row_sum
    mpp_ref[...] = mpp

    part = jnp.sum(jnp.log(v_new))
    prev = jnp.where(b == 0, 0.0, acc_ref[0])
    acc_ref[0] = prev + part

    @pl.when(b == NBLK - 1)
    def _():
        val = -(TEMP / BASE_TEMP) * acc_ref[0] / BSZ
        loss_ref[...] = jnp.reshape(val, (1, 1))


@functools.cache
def _make_dense(interpret=False):
    col = lambda b: (b, 0)
    full = lambda b: (0, 0)
    return pl.pallas_call(
        _dense_body,
        grid=(NBLK,),
        in_specs=[
            pl.BlockSpec((BLK, DFEAT), col),
            pl.BlockSpec((DFEAT, NCON), full),
            pl.BlockSpec((BLK, DFEAT), col),
            pl.BlockSpec((BLK, 1), col),
            pl.BlockSpec((BLK, 1), col),
            pl.BlockSpec((BLK, 1), col),
            pl.BlockSpec((BLK, 1), col),
            pl.BlockSpec((BLK, 1), col),
            pl.BlockSpec((1, BSZ), full),
        ],
        out_specs=[
            pl.BlockSpec((BLK, 1), col),
            pl.BlockSpec((BLK, 1), col),
            pl.BlockSpec((BLK, 1), col),
            pl.BlockSpec((BLK, 1), col),
            pl.BlockSpec((BLK, 1), col),
            pl.BlockSpec((1, 1), full),
        ],
        out_shape=[
            jax.ShapeDtypeStruct((BSZ, 1), jnp.float32),
            jax.ShapeDtypeStruct((BSZ, 1), jnp.float32),
            jax.ShapeDtypeStruct((BSZ, 1), jnp.float32),
            jax.ShapeDtypeStruct((BSZ, 1), jnp.float32),
            jax.ShapeDtypeStruct((BSZ, 1), jnp.int32),
            jax.ShapeDtypeStruct((1, 1), jnp.float32),
        ],
        scratch_shapes=[pltpu.SMEM((1,), jnp.float32)],
        interpret=interpret,
    )


@functools.cache
def _make_sc_gather():
    mesh = plsc.VectorSubcoreMesh(core_axis_name="c", subcore_axis_name="s",
                                  num_cores=NC, num_subcores=NS)

    @functools.partial(
        pl.kernel, mesh=mesh,
        out_type=[jax.ShapeDtypeStruct((BSZ,), jnp.float32)] * 4,
        scratch_types=[pltpu.VMEM((G_CHUNK,), jnp.int32)]
        + [pltpu.VMEM((G_CHUNK,), jnp.float32)] * 4
        + [pltpu.SemaphoreType.DMA],
    )
    def sc_gather(idx_hbm, u_hbm, f1_hbm, v_hbm, f2_hbm,
                  ug_out, f1g_out, vg_out, f2g_out,
                  idx_v, b0, b1, b2, b3, sem):
        wid = lax.axis_index("s") * NC + lax.axis_index("c")
        base = wid * G_CHUNK
        pltpu.sync_copy(idx_hbm.at[pl.ds(base, G_CHUNK)], idx_v)
        c0 = pltpu.async_copy(u_hbm.at[idx_v], b0, sem)
        c1 = pltpu.async_copy(f1_hbm.at[idx_v], b1, sem)
        c2 = pltpu.async_copy(v_hbm.at[idx_v], b2, sem)
        c3 = pltpu.async_copy(f2_hbm.at[idx_v], b3, sem)
        c0.wait(); c1.wait(); c2.wait(); c3.wait()
        pltpu.sync_copy(b0, ug_out.at[pl.ds(base, G_CHUNK)])
        pltpu.sync_copy(b1, f1g_out.at[pl.ds(base, G_CHUNK)])
        pltpu.sync_copy(b2, vg_out.at[pl.ds(base, G_CHUNK)])
        pltpu.sync_copy(b3, f2g_out.at[pl.ds(base, G_CHUNK)])

    return sc_gather


@functools.cache
def _make_sc_scatter():
    mesh = plsc.VectorSubcoreMesh(core_axis_name="c", subcore_axis_name="s",
                                  num_cores=NC, num_subcores=NS)

    @functools.partial(
        pl.kernel, mesh=mesh,
        out_type=[jax.ShapeDtypeStruct((N_MEM,), jnp.float32)] * 4,
        compiler_params=pltpu.CompilerParams(needs_layout_passes=False),
        scratch_types=[
            pltpu.VMEM((SHARD,), jnp.float32),
            pltpu.VMEM((SHARD,), jnp.float32),
            pltpu.VMEM((BSZ,), jnp.int32),
            pltpu.VMEM((BSZ,), jnp.float32),
            pltpu.VMEM((BSZ,), jnp.float32),
            pltpu.SemaphoreType.DMA,
        ],
    )
    def sc_scatter(u_hbm, f1_hbm, v_hbm, f2_hbm, idx_hbm,
                   unew_hbm, rsum_hbm, vnew_hbm, mpp_hbm,
                   u_out, f1_out, v_out, f2_out,
                   t0_v, t1_v, idx_v, val0_v, val1_v, sem):
        cid = lax.axis_index("c")
        sid = lax.axis_index("s")
        lo = sid * SHARD

        def stage(src0, src1):
            cs = [pltpu.async_copy(idx_hbm, idx_v, sem),
                  pltpu.async_copy(src0, val0_v, sem),
                  pltpu.async_copy(src1, val1_v, sem)]
            return cs

        def shards_in(b0, b1):
            @pl.when(sid < NS - 1)
            def _():
                pltpu.sync_copy(b0.at[pl.ds(sid * SHARD, SHARD)], t0_v)
                pltpu.sync_copy(b1.at[pl.ds(sid * SHARD, SHARD)], t1_v)

            @pl.when(sid == NS - 1)
            def _():
                pltpu.sync_copy(b0.at[pl.ds(LAST_START, LAST_N)],
                                t0_v.at[pl.ds(0, LAST_N)])
                pltpu.sync_copy(b1.at[pl.ds(LAST_START, LAST_N)],
                                t1_v.at[pl.ds(0, LAST_N)])

        def shards_out(o0, o1):
            @pl.when(sid < NS - 1)
            def _():
                pltpu.sync_copy(t0_v, o0.at[pl.ds(sid * SHARD, SHARD)])
                pltpu.sync_copy(t1_v, o1.at[pl.ds(sid * SHARD, SHARD)])

            @pl.when(sid == NS - 1)
            def _():
                pltpu.sync_copy(t0_v.at[pl.ds(0, LAST_N)],
                                o0.at[pl.ds(LAST_START, LAST_N)])
                pltpu.sync_copy(t1_v.at[pl.ds(0, LAST_N)],
                                o1.at[pl.ds(LAST_START, LAST_N)])

        def apply_updates():
            def body(k, _):
                iv = idx_v[pl.ds(k * LANES, LANES)]
                msk = (iv >= lo) & (iv < lo + SHARD)
                li = jnp.where(msk, iv - lo, 0)
                plsc.store_scatter(t0_v, [li],
                                   val0_v[pl.ds(k * LANES, LANES)], mask=msk)
                plsc.store_scatter(t1_v, [li],
                                   val1_v[pl.ds(k * LANES, LANES)], mask=msk)
                return 0
            lax.fori_loop(0, BSZ // LANES, body, 0)

        @pl.when(cid == 0)
        def _():
            cs = stage(unew_hbm, rsum_hbm)
            shards_in(u_hbm, f1_hbm)
            for c in cs:
                c.wait()
            apply_updates()
            shards_out(u_out, f1_out)

        @pl.when(cid == 1)
        def _():
            cs = stage(vnew_hbm, mpp_hbm)
            shards_in(v_hbm, f2_hbm)
            for c in cs:
                c.wait()
            apply_updates()
            shards_out(v_out, f2_out)

    return sc_scatter


def kernel(features, index, u, v, f1_w, f2_w):
    anchor = features[:, 0, :]
    ct = jnp.concatenate([anchor, features[:, 1, :]], axis=0).T
    idx = index.astype(jnp.int32)
    u_flat = u.reshape(N_MEM)
    f1_flat = f1_w.reshape(N_MEM)

    ug, f1g, vg, f2g = _make_sc_gather()(idx, u_flat, f1_flat, v, f2_w)

    unew, vnew, rsum, mpp, idxs, loss = _make_dense()(
        anchor, ct, features[:, 1, :],
        idx.reshape(BSZ, 1),
        ug.reshape(BSZ, 1), f1g.reshape(BSZ, 1),
        vg.reshape(BSZ, 1), f2g.reshape(BSZ, 1),
        ug.reshape(1, BSZ),
    )

    u_out, f1_out, v_out, f2_out = _make_sc_scatter()(
        u_flat, f1_flat, v, f2_w, idxs.reshape(BSZ),
        unew.reshape(BSZ), rsum.reshape(BSZ),
        vnew.reshape(BSZ), mpp.reshape(BSZ),
    )

    return (loss[0, 0],
            u_out.reshape(N_MEM, 1),
            v_out,
            f1_out.reshape(N_MEM, 1),
            f2_out)


# speedup vs baseline: 1.1809x; 1.0071x over previous
import functools

import jax
import jax.numpy as jnp
from jax import lax
from jax.experimental import pallas as pl
from jax.experimental.pallas import tpu as pltpu
from jax.experimental.pallas import tpu_sc as plsc

TEMP = 0.07
BASE_TEMP = 0.07
N_MEM = 50020
BSZ = 4096
DFEAT = 64
NCON = 2 * BSZ

BLK = 1024
NBLK = BSZ // BLK
DUMMY = 1 << 20

NC, NS = 2, 16
NW = NC * NS
G_CHUNK = BSZ // NW
LANES = 16
SHARD = 3128
LAST_START = (NS - 1) * SHARD
LAST_N = N_MEM - LAST_START


def _dense_body(a_ref, ct_ref, b_ref, idxc_ref, ug_ref, f1_ref, vg_ref,
                f2_ref, ugall_ref, unew_ref, vnew_ref, rsum_ref, mpp_ref,
                idxs_ref, loss_ref, acc_ref):
    b = pl.program_id(0)
    a = a_ref[...] * (1.0 / TEMP)
    logits = jnp.dot(a, ct_ref[...], preferred_element_type=jnp.float32)
    m = jnp.max(logits, axis=1, keepdims=True)
    e = jnp.exp(logits - m)

    col = lax.broadcasted_iota(jnp.int32, (BLK, NCON), 1)
    rowg = b * BLK + lax.broadcasted_iota(jnp.int32, (BLK, NCON), 0)
    row_sum = jnp.sum(jnp.where(col == rowg, 0.0, e), axis=1, keepdims=True)
    pos_l = jnp.sum(a * b_ref[...], axis=1, keepdims=True)
    pos_e = jnp.exp(pos_l - m)

    gs = jnp.sum(ugall_ref[...])
    gamma = jnp.where(gs == 0.0, 1.0, 0.9)
    omg = 1.0 - gamma
    u_new = omg * (ug_ref[...] - f1_ref[...]) + row_sum
    mpp = pos_e / u_new
    v_new = omg * (vg_ref[...] - f2_ref[...]) + mpp

    idxb = idxc_ref[...]
    lane = lax.broadcasted_iota(jnp.int32, (BLK, 1), 0) % LANES
    loser = jnp.zeros((BLK, 1), jnp.bool_)
    for t in range(1, LANES):
        nb = jnp.concatenate(
            [idxb[t:], jnp.full((t, 1), -1, jnp.int32)], axis=0)
        loser = loser | ((idxb == nb) & (lane < LANES - t))
    idxs_ref[...] = jnp.where(loser, DUMMY, idxb)

    unew_ref[...] = u_new
    vnew_ref[...] = v_new
    rsum_ref[...] = row_sum
    mpp_ref[...] = mpp

    part = jnp.sum(jnp.log(v_new))
    prev = jnp.where(b == 0, 0.0, acc_ref[0])
    acc_ref[0] = prev + part

    @pl.when(b == NBLK - 1)
    def _():
        val = -(TEMP / BASE_TEMP) * acc_ref[0] / BSZ
        loss_ref[...] = jnp.reshape(val, (1, 1))


@functools.cache
def _make_dense(interpret=False):
    col = lambda b: (b, 0)
    full = lambda b: (0, 0)
    return pl.pallas_call(
        _dense_body,
        grid=(NBLK,),
        in_specs=[
            pl.BlockSpec((BLK, DFEAT), col),
            pl.BlockSpec((DFEAT, NCON), full),
            pl.BlockSpec((BLK, DFEAT), col),
            pl.BlockSpec((BLK, 1), col),
            pl.BlockSpec((BLK, 1), col),
            pl.BlockSpec((BLK, 1), col),
            pl.BlockSpec((BLK, 1), col),
            pl.BlockSpec((BLK, 1), col),
            pl.BlockSpec((1, BSZ), full),
        ],
        out_specs=[
            pl.BlockSpec((BLK, 1), col),
            pl.BlockSpec((BLK, 1), col),
            pl.BlockSpec((BLK, 1), col),
            pl.BlockSpec((BLK, 1), col),
            pl.BlockSpec((BLK, 1), col),
            pl.BlockSpec((1, 1), full),
        ],
        out_shape=[
            jax.ShapeDtypeStruct((BSZ, 1), jnp.float32),
            jax.ShapeDtypeStruct((BSZ, 1), jnp.float32),
            jax.ShapeDtypeStruct((BSZ, 1), jnp.float32),
            jax.ShapeDtypeStruct((BSZ, 1), jnp.float32),
            jax.ShapeDtypeStruct((BSZ, 1), jnp.int32),
            jax.ShapeDtypeStruct((1, 1), jnp.float32),
        ],
        scratch_shapes=[pltpu.SMEM((1,), jnp.float32)],
        interpret=interpret,
    )


@functools.cache
def _make_sc_gather():
    mesh = plsc.VectorSubcoreMesh(core_axis_name="c", subcore_axis_name="s",
                                  num_cores=NC, num_subcores=NS)

    @functools.partial(
        pl.kernel, mesh=mesh,
        out_type=[jax.ShapeDtypeStruct((BSZ,), jnp.float32)] * 4,
        scratch_types=[pltpu.VMEM((G_CHUNK,), jnp.int32)]
        + [pltpu.VMEM((G_CHUNK,), jnp.float32)] * 4
        + [pltpu.SemaphoreType.DMA],
    )
    def sc_gather(idx_hbm, u_hbm, f1_hbm, v_hbm, f2_hbm,
                  ug_out, f1g_out, vg_out, f2g_out,
                  idx_v, b0, b1, b2, b3, sem):
        wid = lax.axis_index("s") * NC + lax.axis_index("c")
        base = wid * G_CHUNK
        pltpu.sync_copy(idx_hbm.at[pl.ds(base, G_CHUNK)], idx_v)
        c0 = pltpu.async_copy(u_hbm.at[idx_v], b0, sem)
        c1 = pltpu.async_copy(f1_hbm.at[idx_v], b1, sem)
        c2 = pltpu.async_copy(v_hbm.at[idx_v], b2, sem)
        c3 = pltpu.async_copy(f2_hbm.at[idx_v], b3, sem)
        c0.wait(); c1.wait(); c2.wait(); c3.wait()
        pltpu.sync_copy(b0, ug_out.at[pl.ds(base, G_CHUNK)])
        pltpu.sync_copy(b1, f1g_out.at[pl.ds(base, G_CHUNK)])
        pltpu.sync_copy(b2, vg_out.at[pl.ds(base, G_CHUNK)])
        pltpu.sync_copy(b3, f2g_out.at[pl.ds(base, G_CHUNK)])

    return sc_gather


@functools.cache
def _make_sc_scatter():
    mesh = plsc.VectorSubcoreMesh(core_axis_name="c", subcore_axis_name="s",
                                  num_cores=NC, num_subcores=NS)

    @functools.partial(
        pl.kernel, mesh=mesh,
        out_type=[jax.ShapeDtypeStruct((N_MEM,), jnp.float32)] * 4,
        compiler_params=pltpu.CompilerParams(needs_layout_passes=False),
        scratch_types=[
            pltpu.VMEM((SHARD,), jnp.float32),
            pltpu.VMEM((SHARD,), jnp.float32),
            pltpu.VMEM((BSZ,), jnp.int32),
            pltpu.VMEM((BSZ,), jnp.float32),
            pltpu.VMEM((BSZ,), jnp.float32),
            pltpu.SemaphoreType.DMA,
        ],
    )
    def sc_scatter(u_hbm, f1_hbm, v_hbm, f2_hbm, idx_hbm,
                   unew_hbm, rsum_hbm, vnew_hbm, mpp_hbm,
                   u_out, f1_out, v_out, f2_out,
                   t0_v, t1_v, idx_v, val0_v, val1_v, sem):
        cid = lax.axis_index("c")
        sid = lax.axis_index("s")
        lo = sid * SHARD

        def stage(src0, src1):
            cs = [pltpu.async_copy(idx_hbm, idx_v, sem),
                  pltpu.async_copy(src0, val0_v, sem),
                  pltpu.async_copy(src1, val1_v, sem)]
            return cs

        def shards_in(b0, b1):
            @pl.when(sid < NS - 1)
            def _():
                pltpu.sync_copy(b0.at[pl.ds(sid * SHARD, SHARD)], t0_v)
                pltpu.sync_copy(b1.at[pl.ds(sid * SHARD, SHARD)], t1_v)

            @pl.when(sid == NS - 1)
            def _():
                pltpu.sync_copy(b0.at[pl.ds(LAST_START, LAST_N)],
                                t0_v.at[pl.ds(0, LAST_N)])
                pltpu.sync_copy(b1.at[pl.ds(LAST_START, LAST_N)],
                                t1_v.at[pl.ds(0, LAST_N)])

        def shards_out(o0, o1):
            @pl.when(sid < NS - 1)
            def _():
                pltpu.sync_copy(t0_v, o0.at[pl.ds(sid * SHARD, SHARD)])
                pltpu.sync_copy(t1_v, o1.at[pl.ds(sid * SHARD, SHARD)])

            @pl.when(sid == NS - 1)
            def _():
                pltpu.sync_copy(t0_v.at[pl.ds(0, LAST_N)],
                                o0.at[pl.ds(LAST_START, LAST_N)])
                pltpu.sync_copy(t1_v.at[pl.ds(0, LAST_N)],
                                o1.at[pl.ds(LAST_START, LAST_N)])

        def apply_updates():
            def body(k, _):
                iv = idx_v[pl.ds(k * LANES, LANES)]
                msk = (iv >= lo) & (iv < lo + SHARD)
                li = jnp.where(msk, iv - lo, 0)
                plsc.store_scatter(t0_v, [li],
                                   val0_v[pl.ds(k * LANES, LANES)], mask=msk)
                plsc.store_scatter(t1_v, [li],
                                   val1_v[pl.ds(k * LANES, LANES)], mask=msk)
                return 0
            lax.fori_loop(0, BSZ // LANES, body, 0)

        @pl.when(cid == 0)
        def _():
            cs = stage(unew_hbm, rsum_hbm)
            shards_in(u_hbm, f1_hbm)
            for c in cs:
                c.wait()
            apply_updates()
            shards_out(u_out, f1_out)

        @pl.when(cid == 1)
        def _():
            cs = stage(vnew_hbm, mpp_hbm)
            shards_in(v_hbm, f2_hbm)
            for c in cs:
                c.wait()
            apply_updates()
            shards_out(v_out, f2_out)

    return sc_scatter


def kernel(features, index, u, v, f1_w, f2_w):
    anchor = features[:, 0, :]
    ct = jnp.concatenate([anchor, features[:, 1, :]], axis=0).T
    idx = index.astype(jnp.int32)
    u_flat = u.reshape(N_MEM)
    f1_flat = f1_w.reshape(N_MEM)

    ug, f1g, vg, f2g = _make_sc_gather()(idx, u_flat, f1_flat, v, f2_w)

    unew, vnew, rsum, mpp, idxs, loss = _make_dense()(
        anchor, ct, features[:, 1, :],
        idx.reshape(BSZ, 1),
        ug.reshape(BSZ, 1), f1g.reshape(BSZ, 1),
        vg.reshape(BSZ, 1), f2g.reshape(BSZ, 1),
        ug.reshape(1, BSZ),
    )

    u_out, f1_out, v_out, f2_out = _make_sc_scatter()(
        u_flat, f1_flat, v, f2_w, idxs.reshape(BSZ),
        unew.reshape(BSZ), rsum.reshape(BSZ),
        vnew.reshape(BSZ), mpp.reshape(BSZ),
    )

    return (loss[0, 0],
            u_out.reshape(N_MEM, 1),
            v_out,
            f1_out.reshape(N_MEM, 1),
            f2_out)
